# baseline (speedup 1.0000x reference)
import jax
import jax.numpy as jnp
from jax import lax
from jax.experimental import pallas as pl
from jax.experimental.pallas import tpu as pltpu
from jax.experimental.pallas import tpu_sc as plsc

NUM_EMB = 1000000
G_DIM = 64
R_DIM = 64
OUT_DIM = G_DIM + R_DIM
BATCH = 4096
NB_WORDS = 50
B_TOTAL = BATCH * NB_WORDS

NC = 2
NS = 16
NW = NC * NS
B_PER_W = B_TOTAL // NW
CHUNK = 128
N_CHUNKS = B_PER_W // CHUNK


def _emb_body(xp_hbm, xh_hbm, g_hbm, r_hbm, out_hbm,
              idx_v, hoff_v, gbuf, rbuf, comb, gsem, rsem):
    wid = lax.axis_index("s") * NC + lax.axis_index("c")
    base = wid * B_PER_W
    lane = lax.iota(jnp.int32, 16)

    def chunk_body(c, _):
        off = base + c * CHUNK
        pltpu.sync_copy(xp_hbm.at[pl.ds(off, CHUNK)], idx_v)
        pltpu.sync_copy(xh_hbm.at[pl.ds(off, CHUNK)], hoff_v)
        cg = pltpu.async_copy(g_hbm.at[idx_v], gbuf, gsem)
        cr = pltpu.async_copy(r_hbm.at[idx_v], rbuf, rsem)
        cg.wait()
        cr.wait()

        def row_body(i, _):
            row = jnp.full((16,), i, jnp.int32)
            hvec = plsc.load_gather(hoff_v, [row]) + lane
            for k in range(0, G_DIM, 16):
                comb[i, pl.ds(k, 16)] = plsc.load_gather(gbuf, [row, hvec + k])
            for k in range(0, R_DIM, 16):
                comb[i, pl.ds(G_DIM + k, 16)] = plsc.load_gather(
                    rbuf, [row, hvec + k])
            return 0

        lax.fori_loop(0, CHUNK, row_body, 0)
        pltpu.sync_copy(comb, out_hbm.at[pl.ds(off, CHUNK), :])
        return 0

    lax.fori_loop(0, N_CHUNKS, chunk_body, 0)


def _emb_call(xp_flat, xh_flat, glove_pairs, rand_pairs):
    kern = pl.kernel(
        _emb_body,
        out_type=jax.ShapeDtypeStruct((B_TOTAL, OUT_DIM), jnp.float32),
        mesh=plsc.VectorSubcoreMesh(core_axis_name="c", subcore_axis_name="s"),
        compiler_params=pltpu.CompilerParams(needs_layout_passes=False),
        scratch_types=[
            pltpu.VMEM((CHUNK,), jnp.int32),
            pltpu.VMEM((CHUNK,), jnp.int32),
            pltpu.VMEM((CHUNK, 128), jnp.float32),
            pltpu.VMEM((CHUNK, 128), jnp.float32),
            pltpu.VMEM((CHUNK, OUT_DIM), jnp.float32),
            pltpu.SemaphoreType.DMA,
            pltpu.SemaphoreType.DMA,
        ],
    )
    return kern(xp_flat, xh_flat, glove_pairs, rand_pairs)


def kernel(x, glove_weight, rand_weight):
    x_flat = x.reshape(B_TOTAL)
    xp_flat = x_flat >> 1
    xh_flat = (x_flat & 1) * G_DIM
    glove_pairs = glove_weight.reshape(NUM_EMB // 2, 2 * G_DIM)
    rand_pairs = rand_weight.reshape(NUM_EMB // 2, 2 * R_DIM)
    out = _emb_call(xp_flat, xh_flat, glove_pairs, rand_pairs)
    return out.reshape(BATCH, NB_WORDS, OUT_DIM)


# speedup vs baseline: 1.4104x; 1.4104x over previous
import jax
import jax.numpy as jnp
from jax import lax
from jax.experimental import pallas as pl
from jax.experimental.pallas import tpu as pltpu
from jax.experimental.pallas import tpu_sc as plsc

NUM_EMB = 1000000
G_DIM = 64
R_DIM = 64
OUT_DIM = G_DIM + R_DIM
BATCH = 4096
NB_WORDS = 50
B_TOTAL = BATCH * NB_WORDS

NC = 2
NS = 16
NW = NC * NS
B_PER_W = B_TOTAL // NW
GROUP = 128
N_GROUPS = B_PER_W // GROUP

JB = 256
JB_SHIFT = 8
J0_MAX = 1000064 - JB


def _emb_body(sidx_hbm, spos_hbm, g_hbm, r_hbm, out_hbm,
              sidx_v, spos_v, gblk, rblk, comb,
              gpsem, rpsem, ssem):
    wid = lax.axis_index("s") * NC + lax.axis_index("c")
    lane = lax.iota(jnp.int32, 16)

    def window(b):
        return jnp.minimum(b * JB, J0_MAX)

    pltpu.sync_copy(sidx_hbm.at[wid], sidx_v.at[pl.ds(0, B_PER_W)])
    pltpu.sync_copy(spos_hbm.at[wid], spos_v)

    b0 = sidx_v[pl.ds(0, 16)][0] >> JB_SHIFT
    pltpu.async_copy(g_hbm.at[:, pl.ds(window(b0), JB)], gblk.at[0], gpsem)
    pltpu.async_copy(r_hbm.at[:, pl.ds(window(b0), JB)], rblk.at[0], rpsem)

    def hit_body(sl, carry):
        cur_b, pref_b, par = carry
        px = par ^ 1
        j = sidx_v[pl.ds(sl, 16)][0]
        b = j >> JB_SHIFT
        reload = b != cur_b

        @pl.when(reload)
        def _():
            pltpu.make_async_copy(
                g_hbm.at[:, pl.ds(0, JB)], gblk.at[px], gpsem).wait()
            pltpu.make_async_copy(
                r_hbm.at[:, pl.ds(0, JB)], rblk.at[px], rpsem).wait()

        @pl.when(jnp.logical_and(reload, b != pref_b))
        def _():
            pltpu.sync_copy(g_hbm.at[:, pl.ds(window(b), JB)], gblk.at[px])
            pltpu.sync_copy(r_hbm.at[:, pl.ds(window(b), JB)], rblk.at[px])

        @pl.when(reload)
        def _():
            pltpu.async_copy(
                g_hbm.at[:, pl.ds(window(b + 1), JB)], gblk.at[par], gpsem)
            pltpu.async_copy(
                r_hbm.at[:, pl.ds(window(b + 1), JB)], rblk.at[par], rpsem)

        new_par = jnp.where(reload, px, par)
        colv = jnp.full((16,), j - window(b), jnp.int32)
        parv = jnp.full((16,), new_par, jnp.int32)
        rowi = sl & (GROUP - 1)
        for gi in range(0, G_DIM, 16):
            comb[rowi, pl.ds(gi, 16)] = plsc.load_gather(
                gblk, [parv, lane + gi, colv])
        for gi in range(0, R_DIM, 16):
            comb[rowi, pl.ds(G_DIM + gi, 16)] = plsc.load_gather(
                rblk, [parv, lane + gi, colv])

        @pl.when(rowi == GROUP - 1)
        def _():
            pltpu.async_copy(
                comb, out_hbm.at[spos_v.at[sl >> 7]], ssem).wait()

        return (jnp.where(reload, b, cur_b),
                jnp.where(reload, b + 1, pref_b),
                new_par)

    lax.fori_loop(0, B_PER_W, hit_body,
                  (jnp.int32(-1), b0, jnp.int32(1)))

    pltpu.make_async_copy(
        g_hbm.at[:, pl.ds(0, JB)], gblk.at[0], gpsem).wait()
    pltpu.make_async_copy(
        r_hbm.at[:, pl.ds(0, JB)], rblk.at[0], rpsem).wait()


def _emb_call(sidx2, spos3, glove_t, rand_t):
    kern = pl.kernel(
        _emb_body,
        out_type=jax.ShapeDtypeStruct((B_TOTAL, OUT_DIM), jnp.float32),
        mesh=plsc.VectorSubcoreMesh(core_axis_name="c", subcore_axis_name="s"),
        compiler_params=pltpu.CompilerParams(needs_layout_passes=False),
        scratch_types=[
            pltpu.VMEM((B_PER_W + 16,), jnp.int32),
            pltpu.VMEM((N_GROUPS, GROUP), jnp.int32),
            pltpu.VMEM((2, G_DIM, JB), jnp.float32),
            pltpu.VMEM((2, R_DIM, JB), jnp.float32),
            pltpu.VMEM((GROUP, OUT_DIM), jnp.float32),
            pltpu.SemaphoreType.DMA,
            pltpu.SemaphoreType.DMA,
            pltpu.SemaphoreType.DMA,
        ],
    )
    return kern(sidx2, spos3, glove_t, rand_t)


def kernel(x, glove_weight, rand_weight):
    x_flat = x.reshape(B_TOTAL)
    pos = lax.iota(jnp.int32, B_TOTAL)
    sidx, spos = lax.sort([x_flat, pos], num_keys=1)
    sidx2 = sidx.reshape(NW, B_PER_W)
    spos3 = spos.reshape(NW, N_GROUPS, GROUP)
    out = _emb_call(sidx2, spos3, glove_weight.T, rand_weight.T)
    return out.reshape(BATCH, NB_WORDS, OUT_DIM)
